# baseline (speedup 1.0000x reference)
import functools

import jax
import jax.numpy as jnp
from jax import lax
from jax.experimental import pallas as pl
from jax.experimental.pallas import tpu as pltpu
from jax.experimental.pallas import tpu_sc as plsc

L = 16
NW = 32
CPW = 12
CROP = 14
CELLS = CROP * CROP
GROUPS = 13
BLK = 4
NBLK = 128
SLAB = CELLS * BLK * CPW

_mesh = plsc.VectorSubcoreMesh(
    core_axis_name="c", subcore_axis_name="s", num_cores=2, num_subcores=16
)


@functools.partial(
    pl.kernel,
    mesh=_mesh,
    out_type=jax.ShapeDtypeStruct((CELLS, 512, NW, CPW), jnp.float32),
    scratch_types=[
        pltpu.VMEM((CPW, 4096), jnp.float32),
        pltpu.VMEM((2048, L), jnp.float32),
        pltpu.VMEM((5 * 512,), jnp.float32),
        pltpu.VMEM((2, CELLS, BLK, CPW), jnp.float32),
        pltpu.SemaphoreType.DMA,
    ],
    compiler_params=pltpu.CompilerParams(
        use_tc_tiling_on_sc=False, needs_layout_passes=False
    ),
)
def _roi_align_sc(tab_hbm, rois_hbm, out_hbm, tab_vm, stage_vm, rois_vm, outbuf, sem):
    cid = lax.axis_index("c")
    sid = lax.axis_index("s")
    wid = cid * 16 + sid
    c0 = wid * CPW

    iota = lax.iota(jnp.int32, L)

    pltpu.sync_copy(rois_hbm, rois_vm)
    al = (c0 // 8) * 8
    sub = c0 - al
    for half in range(2):
        pltpu.sync_copy(
            tab_hbm.at[pl.ds(half * 2048, 2048), pl.ds(al, L)], stage_vm
        )

        def tr_body(g, _, half=half):
            rbase = g * L
            for c in range(CPW):
                col = plsc.load_gather(
                    stage_vm, [rbase + iota, jnp.full((L,), c, jnp.int32) + sub]
                )
                tab_vm[c, pl.ds(half * 2048 + rbase, L)] = col
            return 0

        lax.fori_loop(0, 2048 // L, tr_body, 0)
    tab_c = [tab_vm.at[c] for c in range(CPW)]

    def blk_body(t, _):
        buf = t % 2
        obuf = outbuf.at[buf]

        @pl.when(t >= 2)
        def _wait():
            pltpu.make_async_copy(
                outbuf.at[0],
                out_hbm.at[:, pl.ds(0, BLK), wid, :],
                sem,
            ).wait()

        def roi_body(mo, _):
            m = t * BLK + mo
            msplat = jnp.full((L,), m, jnp.int32)
            mosplat = jnp.full((L,), mo, jnp.int32)
            b_f = plsc.load_gather(rois_vm, [msplat])
            x1v = plsc.load_gather(rois_vm, [msplat + 512])
            y1v = plsc.load_gather(rois_vm, [msplat + 1024])
            dxv = plsc.load_gather(rois_vm, [msplat + 1536])
            dyv = plsc.load_gather(rois_vm, [msplat + 2048])
            bb = b_f.astype(jnp.int32) * 1024

            def grp_body(g, _):
                q = jnp.full((L,), g * L, jnp.int32) + iota
                i_ = jnp.right_shift(q * 4682, 16)
                j_ = q - i_ * CROP
                ys = y1v + dyv * i_.astype(jnp.float32)
                xs = x1v + dxv * j_.astype(jnp.float32)
                y0 = jnp.minimum(ys.astype(jnp.int32), 31)
                x0 = jnp.minimum(xs.astype(jnp.int32), 31)
                wy = ys - y0.astype(jnp.float32)
                wx = xs - x0.astype(jnp.float32)
                y1c = jnp.minimum(y0 + 1, 31)
                x1c = jnp.minimum(x0 + 1, 31)
                row0 = bb + y0 * 32
                row1 = bb + y1c * 32
                b00 = row0 + x0
                b01 = row0 + x1c
                b10 = row1 + x0
                b11 = row1 + x1c
                w11 = wy * wx
                w10 = wy - w11
                w01 = wx - w11
                w00 = (1.0 - wy) - w01
                msk = q < CELLS
                for c3 in range(CPW // 4):
                    loads = []
                    for cc in range(4):
                        tr = tab_c[c3 * 4 + cc]
                        loads.append(
                            (
                                plsc.load_gather(tr, [b00]),
                                plsc.load_gather(tr, [b01]),
                                plsc.load_gather(tr, [b10]),
                                plsc.load_gather(tr, [b11]),
                            )
                        )
                    for cc in range(4):
                        g00, g01, g10, g11 = loads[cc]
                        v = (w00 * g00 + w01 * g01) + (w10 * g10 + w11 * g11)
                        c = c3 * 4 + cc
                        plsc.store_scatter(
                            obuf,
                            [q, mosplat, jnp.full((L,), c, jnp.int32)],
                            v,
                            mask=msk,
                        )
                return 0

            lax.fori_loop(0, GROUPS, grp_body, 0)
            return 0

        lax.fori_loop(0, BLK, roi_body, 0)

        pltpu.async_copy(
            obuf,
            out_hbm.at[:, pl.ds(t * BLK, BLK), wid, :],
            sem,
        )
        return 0

    lax.fori_loop(0, NBLK, blk_body, 0)

    for _ in range(2):
        pltpu.make_async_copy(
            outbuf.at[0],
            out_hbm.at[:, pl.ds(0, BLK), wid, :],
            sem,
        ).wait()


def kernel(feature_map, rois, img_height):
    N, C, H, W = feature_map.shape
    M = rois.shape[0]
    inv = jnp.float32(H) / jnp.asarray(img_height, jnp.float32)
    b = rois[:, 0]
    x1 = rois[:, 2] * inv
    y1 = rois[:, 3] * inv
    dx = (rois[:, 4] - rois[:, 2]) * inv * (1.0 / (CROP - 1))
    dy = (rois[:, 5] - rois[:, 3]) * inv * (1.0 / (CROP - 1))
    rois_p = jnp.concatenate([b, x1, y1, dx, dy])

    tab = feature_map.transpose(0, 2, 3, 1).reshape(N * H * W, C)
    out = _roi_align_sc(tab, rois_p)
    return out.reshape(CROP, CROP, M, C).transpose(2, 3, 0, 1)


# speedup vs baseline: 3.2888x; 3.2888x over previous
import functools

import jax
import jax.numpy as jnp
from jax import lax
from jax.experimental import pallas as pl
from jax.experimental.pallas import tpu as pltpu
from jax.experimental.pallas import tpu_sc as plsc

L = 16
NW = 32
CPW = 12
CROP = 14
CELLS = CROP * CROP
FULLG = CELLS // L
BLK = 8
NBLK = 64

_mesh = plsc.VectorSubcoreMesh(
    core_axis_name="c", subcore_axis_name="s", num_cores=2, num_subcores=16
)


@functools.partial(
    pl.kernel,
    mesh=_mesh,
    out_type=jax.ShapeDtypeStruct((512, 384, CELLS), jnp.float32),
    scratch_types=[
        pltpu.VMEM((CPW, 4096), jnp.float32),
        pltpu.VMEM((2048, L), jnp.float32),
        pltpu.VMEM((5 * 512,), jnp.float32),
        pltpu.VMEM((2, BLK, CPW, CELLS), jnp.float32),
        pltpu.SemaphoreType.DMA,
    ],
    compiler_params=pltpu.CompilerParams(
        use_tc_tiling_on_sc=False, needs_layout_passes=False
    ),
)
def _roi_align_sc(tab_hbm, rois_hbm, out_hbm, tab_vm, stage_vm, rois_vm, outbuf, sem):
    cid = lax.axis_index("c")
    sid = lax.axis_index("s")
    wid = cid * 16 + sid
    c0 = wid * CPW

    iota = lax.iota(jnp.int32, L)

    pltpu.sync_copy(rois_hbm, rois_vm)
    al = (c0 // 8) * 8
    sub = c0 - al
    for half in range(2):
        pltpu.sync_copy(
            tab_hbm.at[pl.ds(half * 2048, 2048), pl.ds(al, L)], stage_vm
        )

        def tr_body(g, _, half=half):
            rbase = g * L
            for c in range(CPW):
                col = plsc.load_gather(
                    stage_vm, [rbase + iota, jnp.full((L,), c, jnp.int32) + sub]
                )
                tab_vm[c, pl.ds(half * 2048 + rbase, L)] = col
            return 0

        lax.fori_loop(0, 2048 // L, tr_body, 0)
    tab_c = [tab_vm.at[c] for c in range(CPW)]

    def blk_body(t, _):
        buf = t % 2

        @pl.when(t >= 2)
        def _wait():
            pltpu.make_async_copy(
                outbuf.at[0],
                out_hbm.at[pl.ds(0, BLK), pl.ds(c0, CPW), :],
                sem,
            ).wait()

        def roi_body(mo, _):
            m = t * BLK + mo
            msplat = jnp.full((L,), m, jnp.int32)
            b_f = plsc.load_gather(rois_vm, [msplat])
            x1v = plsc.load_gather(rois_vm, [msplat + 512])
            y1v = plsc.load_gather(rois_vm, [msplat + 1024])
            dxv = plsc.load_gather(rois_vm, [msplat + 1536])
            dyv = plsc.load_gather(rois_vm, [msplat + 2048])
            bb = b_f.astype(jnp.int32) * 1024

            def cell_group(q, store):
                i_ = jnp.right_shift(q * 4682, 16)
                j_ = q - i_ * CROP
                ys = y1v + dyv * i_.astype(jnp.float32)
                xs = x1v + dxv * j_.astype(jnp.float32)
                y0 = jnp.minimum(ys.astype(jnp.int32), 31)
                x0 = jnp.minimum(xs.astype(jnp.int32), 31)
                wy = ys - y0.astype(jnp.float32)
                wx = xs - x0.astype(jnp.float32)
                y1c = jnp.minimum(y0 + 1, 31)
                x1c = jnp.minimum(x0 + 1, 31)
                row0 = bb + y0 * 32
                row1 = bb + y1c * 32
                b00 = row0 + x0
                b01 = row0 + x1c
                b10 = row1 + x0
                b11 = row1 + x1c
                w11 = wy * wx
                w10 = wy - w11
                w01 = wx - w11
                w00 = (1.0 - wy) - w01
                for c3 in range(CPW // 4):
                    loads = []
                    for cc in range(4):
                        tr = tab_c[c3 * 4 + cc]
                        loads.append(
                            (
                                plsc.load_gather(tr, [b00]),
                                plsc.load_gather(tr, [b01]),
                                plsc.load_gather(tr, [b10]),
                                plsc.load_gather(tr, [b11]),
                            )
                        )
                    for cc in range(4):
                        g00, g01, g10, g11 = loads[cc]
                        v = (w00 * g00 + w01 * g01) + (w10 * g10 + w11 * g11)
                        store(c3 * 4 + cc, v)

            def grp_body(g, _):
                q = jnp.full((L,), g * L, jnp.int32) + iota
                g16 = g * L

                def store(c, v):
                    outbuf[buf, mo, c, pl.ds(g16, L)] = v

                cell_group(q, store)
                return 0

            lax.fori_loop(0, FULLG, grp_body, 0)

            qt = jnp.full((L,), FULLG * L, jnp.int32) + iota
            mt = qt < CELLS

            def store_tail(c, v):
                plsc.store_scatter(outbuf.at[buf, mo, c], [qt], v, mask=mt)

            cell_group(qt, store_tail)
            return 0

        lax.fori_loop(0, BLK, roi_body, 0)

        pltpu.async_copy(
            outbuf.at[buf],
            out_hbm.at[pl.ds(t * BLK, BLK), pl.ds(c0, CPW), :],
            sem,
        )
        return 0

    lax.fori_loop(0, NBLK, blk_body, 0)

    for _ in range(2):
        pltpu.make_async_copy(
            outbuf.at[0],
            out_hbm.at[pl.ds(0, BLK), pl.ds(c0, CPW), :],
            sem,
        ).wait()


def kernel(feature_map, rois, img_height):
    N, C, H, W = feature_map.shape
    M = rois.shape[0]
    inv = jnp.float32(H) / jnp.asarray(img_height, jnp.float32)
    b = rois[:, 0]
    x1 = rois[:, 2] * inv
    y1 = rois[:, 3] * inv
    dx = (rois[:, 4] - rois[:, 2]) * inv * (1.0 / (CROP - 1))
    dy = (rois[:, 5] - rois[:, 3]) * inv * (1.0 / (CROP - 1))
    rois_p = jnp.concatenate([b, x1, y1, dx, dy])

    tab = feature_map.transpose(0, 2, 3, 1).reshape(N * H * W, C)
    out = _roi_align_sc(tab, rois_p)
    return out.reshape(M, C, CROP, CROP)
